# baseline (speedup 1.0000x reference)
import functools

import jax
import jax.numpy as jnp
from jax import lax
from jax.experimental import pallas as pl
from jax.experimental.pallas import tpu as pltpu
from jax.experimental.pallas import tpu_sc as plsc

CHAR_SIZE = 100000
EMB_DIM = 30
PROJ_DIM = 300
BATCH = 16384
SEQ = 200

PAD_D = 32
GROUP = PROJ_DIM // EMB_DIM
NIDX = BATCH * SEQ
ROWS = NIDX // GROUP
KPAD = 384

NC, NS = 2, 16
NW = NC * NS
RPG = 128

P = 16
NIDX_P = NIDX // P
ROWS_P = ROWS // P
NTROW_P = ROWS_P // 8
NLINES_P = ROWS_P * KPAD // PAD_D
N128_P = NLINES_P // 4
PER_W = NIDX_P // NW
BLOCKS_PER_W = PER_W // RPG
K = next(k for k in range(16, 0, -1) if BLOCKS_PER_W % k == 0)
ITERS = BLOCKS_PER_W // K

RB8 = 128
NBLK_P = NTROW_P // RB8


def _sc_gather_body(idx_hbm, lidx_hbm, table_hbm, out_hbm, idx_v, lidx_v,
                    rows_v, gsem, ssem):
    wid = lax.axis_index("s") * NC + lax.axis_index("c")

    def outer(i, carry):
        blk0 = wid * BLOCKS_PER_W + i * K
        pltpu.sync_copy(idx_hbm.at[pl.ds(blk0, K)], idx_v)
        pltpu.sync_copy(lidx_hbm.at[pl.ds(blk0, K)], lidx_v)
        gcps = [
            pltpu.async_copy(
                table_hbm.at[idx_v.at[j]], rows_v.at[pl.ds(j * RPG, RPG)], gsem
            )
            for j in range(K)
        ]
        for cp in gcps:
            cp.wait()
        scps = [
            pltpu.async_copy(
                rows_v.at[pl.ds(j * RPG, RPG)], out_hbm.at[lidx_v.at[j]], ssem
            )
            for j in range(K)
        ]
        for cp in scps:
            cp.wait()
        return carry

    lax.fori_loop(0, ITERS, outer, 0)


@functools.lru_cache(maxsize=None)
def _sc_gather():
    return pl.kernel(
        _sc_gather_body,
        out_type=jax.ShapeDtypeStruct((NLINES_P, PAD_D), jnp.float32),
        mesh=plsc.VectorSubcoreMesh(
            core_axis_name="c", subcore_axis_name="s", num_cores=NC, num_subcores=NS
        ),
        scratch_types=[
            pltpu.VMEM((K, RPG), jnp.int32),
            pltpu.VMEM((K, RPG), jnp.int32),
            pltpu.VMEM((K * RPG, PAD_D), jnp.float32),
            pltpu.SemaphoreType.DMA,
            pltpu.SemaphoreType.DMA,
        ],
        compiler_params=pltpu.CompilerParams(use_tc_tiling_on_sc=False),
    )


def _dest_lines():
    m = jnp.arange(NIDX_P, dtype=jnp.int32)
    r = m // GROUP
    j = m - r * GROUP
    return (r // 8) * 96 + (j // 4) * 32 + (r % 8) * 4 + (j % 4)


def _mm_body(a_ref, w_ref, o_ref):
    a4 = a_ref[...].reshape(RB8, 3, 8, 128)
    acc = None
    for c in range(3):
        ac = a4[:, c].reshape(RB8 * 8, 128)
        if c == 2:
            lanes = lax.broadcasted_iota(jnp.int32, (RB8 * 8, 128), 1)
            ac = jnp.where(lanes < 64, ac, 0.0)
        p = jnp.dot(
            ac,
            w_ref[pl.ds(c * 128, 128), :],
            preferred_element_type=jnp.float32,
        )
        acc = p if acc is None else acc + p
    o_ref[...] = acc


def _mm_body_acc(a_ref, w_ref, prev_ref, o_ref):
    del prev_ref
    _mm_body(a_ref, w_ref, o_ref)


def _project_piece(a, w384, prev, p):
    in_specs = [
        pl.BlockSpec((RB8 * 24, 128), lambda i: (i, 0)),
        pl.BlockSpec((KPAD, PROJ_DIM), lambda i: (0, 0)),
    ]
    out_spec = pl.BlockSpec(
        (RB8 * 8, PROJ_DIM), lambda i, p=p: (i + p * NBLK_P, 0)
    )
    out_shape = jax.ShapeDtypeStruct((ROWS, PROJ_DIM), jnp.float32)
    if prev is None:
        return pl.pallas_call(
            _mm_body, grid=(NBLK_P,), in_specs=in_specs,
            out_specs=out_spec, out_shape=out_shape,
        )(a, w384)
    return pl.pallas_call(
        _mm_body_acc, grid=(NBLK_P,),
        in_specs=in_specs + [pl.BlockSpec(memory_space=pl.ANY)],
        out_specs=out_spec, out_shape=out_shape,
        input_output_aliases={2: 0},
    )(a, w384, prev)


def kernel(X, table, W):
    table_pad = jnp.pad(table, ((0, 0), (0, PAD_D - EMB_DIM)))
    lidx = _dest_lines().reshape(NIDX_P // RPG, RPG)
    wp = jnp.pad(
        W.T.reshape(GROUP, EMB_DIM, PROJ_DIM),
        ((0, 0), (0, PAD_D - EMB_DIM), (0, 0)),
    ).reshape(GROUP * PAD_D, PROJ_DIM)
    w384 = jnp.pad(wp, ((0, KPAD - GROUP * PAD_D), (0, 0)))

    bp = BATCH // P
    packs = []
    for p in range(P):
        xp = lax.slice_in_dim(X, p * bp, (p + 1) * bp)
        idx = xp.reshape(NIDX_P // RPG, RPG).astype(jnp.int32)
        lines = _sc_gather()(idx, lidx, table_pad)
        packs.append(lines.reshape(N128_P, 128))
    out = None
    for p in range(P):
        out = _project_piece(packs[p], w384, out, p)
    return out


# speedup vs baseline: 1.0101x; 1.0101x over previous
import functools

import jax
import jax.numpy as jnp
from jax import lax
from jax.experimental import pallas as pl
from jax.experimental.pallas import tpu as pltpu
from jax.experimental.pallas import tpu_sc as plsc

CHAR_SIZE = 100000
EMB_DIM = 30
PROJ_DIM = 300
BATCH = 16384
SEQ = 200

PAD_D = 32
GROUP = PROJ_DIM // EMB_DIM
NIDX = BATCH * SEQ
ROWS = NIDX // GROUP
KPAD = 384

NC, NS = 2, 16
NW = NC * NS
RPG = 128

P = 8
NIDX_P = NIDX // P
ROWS_P = ROWS // P
NTROW_P = ROWS_P // 8
NLINES_P = ROWS_P * KPAD // PAD_D
N128_P = NLINES_P // 4
PER_W = NIDX_P // NW
BLOCKS_PER_W = PER_W // RPG
K = next(k for k in range(16, 0, -1) if BLOCKS_PER_W % k == 0)
ITERS = BLOCKS_PER_W // K

RB8 = 128
NBLK_P = NTROW_P // RB8


def _sc_gather_body(idx_hbm, lidx_hbm, table_hbm, out_hbm, idx_v, lidx_v,
                    rows_v, gsem, ssem):
    wid = lax.axis_index("s") * NC + lax.axis_index("c")

    def outer(i, carry):
        blk0 = wid * BLOCKS_PER_W + i * K
        pltpu.sync_copy(idx_hbm.at[pl.ds(blk0, K)], idx_v)
        pltpu.sync_copy(lidx_hbm.at[pl.ds(blk0, K)], lidx_v)
        gcps = [
            pltpu.async_copy(
                table_hbm.at[idx_v.at[j]], rows_v.at[pl.ds(j * RPG, RPG)], gsem
            )
            for j in range(K)
        ]
        for cp in gcps:
            cp.wait()
        scps = [
            pltpu.async_copy(
                rows_v.at[pl.ds(j * RPG, RPG)], out_hbm.at[lidx_v.at[j]], ssem
            )
            for j in range(K)
        ]
        for cp in scps:
            cp.wait()
        return carry

    lax.fori_loop(0, ITERS, outer, 0)


@functools.lru_cache(maxsize=None)
def _sc_gather():
    return pl.kernel(
        _sc_gather_body,
        out_type=jax.ShapeDtypeStruct((NLINES_P, PAD_D), jnp.float32),
        mesh=plsc.VectorSubcoreMesh(
            core_axis_name="c", subcore_axis_name="s", num_cores=NC, num_subcores=NS
        ),
        scratch_types=[
            pltpu.VMEM((K, RPG), jnp.int32),
            pltpu.VMEM((K, RPG), jnp.int32),
            pltpu.VMEM((K * RPG, PAD_D), jnp.float32),
            pltpu.SemaphoreType.DMA,
            pltpu.SemaphoreType.DMA,
        ],
        compiler_params=pltpu.CompilerParams(use_tc_tiling_on_sc=False),
    )


def _dest_lines():
    m = jnp.arange(NIDX_P, dtype=jnp.int32)
    r = m // GROUP
    j = m - r * GROUP
    return (r // 8) * 96 + (j // 4) * 32 + (r % 8) * 4 + (j % 4)


def _mm_body(a_ref, w_ref, o_ref):
    a4 = a_ref[...].reshape(RB8, 3, 8, 128)
    acc = None
    for c in range(3):
        ac = a4[:, c].reshape(RB8 * 8, 128)
        if c == 2:
            lanes = lax.broadcasted_iota(jnp.int32, (RB8 * 8, 128), 1)
            ac = jnp.where(lanes < 64, ac, 0.0)
        p = jnp.dot(
            ac,
            w_ref[pl.ds(c * 128, 128), :],
            preferred_element_type=jnp.float32,
        )
        acc = p if acc is None else acc + p
    o_ref[...] = acc


def _mm_body_acc(a_ref, w_ref, prev_ref, o_ref):
    del prev_ref
    _mm_body(a_ref, w_ref, o_ref)


def _project_piece(a, w384, prev, p):
    in_specs = [
        pl.BlockSpec((RB8 * 24, 128), lambda i: (i, 0)),
        pl.BlockSpec((KPAD, PROJ_DIM), lambda i: (0, 0)),
    ]
    out_spec = pl.BlockSpec(
        (RB8 * 8, PROJ_DIM), lambda i, p=p: (i + p * NBLK_P, 0)
    )
    out_shape = jax.ShapeDtypeStruct((ROWS, PROJ_DIM), jnp.float32)
    if prev is None:
        return pl.pallas_call(
            _mm_body, grid=(NBLK_P,), in_specs=in_specs,
            out_specs=out_spec, out_shape=out_shape,
        )(a, w384)
    return pl.pallas_call(
        _mm_body_acc, grid=(NBLK_P,),
        in_specs=in_specs + [pl.BlockSpec(memory_space=pl.ANY)],
        out_specs=out_spec, out_shape=out_shape,
        input_output_aliases={2: 0},
    )(a, w384, prev)


def kernel(X, table, W):
    table_pad = jnp.pad(table, ((0, 0), (0, PAD_D - EMB_DIM)))
    lidx = _dest_lines().reshape(NIDX_P // RPG, RPG)
    wp = jnp.pad(
        W.T.reshape(GROUP, EMB_DIM, PROJ_DIM),
        ((0, 0), (0, PAD_D - EMB_DIM), (0, 0)),
    ).reshape(GROUP * PAD_D, PROJ_DIM)
    w384 = jnp.pad(wp, ((0, KPAD - GROUP * PAD_D), (0, 0)))

    bp = BATCH // P
    packs = []
    for p in range(P):
        xp = lax.slice_in_dim(X, p * bp, (p + 1) * bp)
        idx = xp.reshape(NIDX_P // RPG, RPG).astype(jnp.int32)
        lines = _sc_gather()(idx, lidx, table_pad)
        packs.append(lines.reshape(N128_P, 128))
    out = None
    for p in range(P):
        out = _project_piece(packs[p], w384, out, p)
    return out
